# Initial kernel scaffold; baseline (speedup 1.0000x reference)
import jax
import jax.numpy as jnp
from jax.experimental import pallas as pl


def kernel(x, gate_w, expert_w):
    raise NotImplementedError("write your pallas kernel here")


# speedup vs baseline: 2.5921x; 2.5921x over previous
import jax
import jax.numpy as jnp
from jax.experimental import pallas as pl

_TOKENS = 8192
_D = 32
_E = 4
_BLK = 1024


def _moe_block(x_ref, gwt_ref, wcat_ref, out_ref):
    xb = x_ref[...]
    logits = jnp.dot(xb, gwt_ref[...],
                     preferred_element_type=jnp.float32)
    cols = [logits[:, e:e + 1] for e in range(_E)]
    y = jnp.dot(xb, wcat_ref[...],
                preferred_element_type=jnp.float32)
    acc = jnp.zeros_like(xb)
    for e in range(_E):
        beat = jnp.zeros_like(cols[0], dtype=jnp.int32)
        for f in range(_E):
            if f == e:
                continue
            if f < e:
                b = cols[f] >= cols[e]
            else:
                b = cols[f] > cols[e]
            beat = beat + b.astype(jnp.int32)
        mask = (beat < 2).astype(jnp.float32)
        acc = acc + mask * y[:, e * _D:(e + 1) * _D]
    out_ref[...] = acc


@jax.jit
def kernel(x, gate_w, expert_w):
    gwt = gate_w.T
    wcat = jnp.transpose(expert_w, (2, 0, 1)).reshape(_D, _E * _D)
    grid = (_TOKENS // _BLK,)
    return pl.pallas_call(
        _moe_block,
        grid=grid,
        in_specs=[
            pl.BlockSpec((_BLK, _D), lambda i: (i, 0)),
            pl.BlockSpec((_D, _E), lambda i: (0, 0)),
            pl.BlockSpec((_D, _E * _D), lambda i: (0, 0)),
        ],
        out_specs=pl.BlockSpec((_BLK, _D), lambda i: (i, 0)),
        out_shape=jax.ShapeDtypeStruct((_TOKENS, _D), jnp.float32),
    )(x, gwt, wcat)


# speedup vs baseline: 2.8212x; 1.0884x over previous
import jax
import jax.numpy as jnp
from jax.experimental import pallas as pl

_TOKENS = 8192
_D = 32
_E = 4
_BLK = 1024

_CONTRACT_1_1 = (((1,), (1,)), ((), ()))


def _moe_block(x_ref, gw_ref, wcat_ref, out_ref):
    xb = x_ref[...]
    logits = jax.lax.dot_general(xb, gw_ref[...], _CONTRACT_1_1,
                                 preferred_element_type=jnp.float32)
    y = jax.lax.dot_general(xb, wcat_ref[...], _CONTRACT_1_1,
                            preferred_element_type=jnp.float32)
    cols = [logits[:, e:e + 1] for e in range(_E)]
    acc = jnp.zeros_like(xb)
    for e in range(_E):
        beat = jnp.zeros_like(cols[0], dtype=jnp.int32)
        for f in range(_E):
            if f == e:
                continue
            if f < e:
                b = cols[f] >= cols[e]
            else:
                b = cols[f] > cols[e]
            beat = beat + b.astype(jnp.int32)
        mask = (beat < 2).astype(jnp.float32)
        acc = acc + mask * y[:, e * _D:(e + 1) * _D]
    out_ref[...] = acc


@jax.jit
def kernel(x, gate_w, expert_w):
    wcat = expert_w.reshape(_E * _D, _D)
    grid = (_TOKENS // _BLK,)
    return pl.pallas_call(
        _moe_block,
        grid=grid,
        in_specs=[
            pl.BlockSpec((_BLK, _D), lambda i: (i, 0)),
            pl.BlockSpec((_E, _D), lambda i: (0, 0)),
            pl.BlockSpec((_E * _D, _D), lambda i: (0, 0)),
        ],
        out_specs=pl.BlockSpec((_BLK, _D), lambda i: (i, 0)),
        out_shape=jax.ShapeDtypeStruct((_TOKENS, _D), jnp.float32),
    )(x, gate_w, wcat)


# speedup vs baseline: 5.2412x; 1.8578x over previous
import numpy as np
import jax
import jax.numpy as jnp
from jax.experimental import pallas as pl

_TOKENS = 8192
_D = 32
_E = 4
_BLK = 1024

_CONTRACT_1_1 = (((1,), (1,)), ((), ()))
_CONTRACT_1_0 = (((1,), (0,)), ((), ()))

_PAIRS = [(e, f) for e in range(_E) for f in range(e + 1, _E)]

_DIF = np.zeros((_E, len(_PAIRS)), np.float32)
_M = np.zeros((len(_PAIRS), _E), np.float32)
for p, (e, f) in enumerate(_PAIRS):
    _DIF[f, p] = 1.0
    _DIF[e, p] = -1.0
    _M[p, e] = 1.0
    _M[p, f] = -1.0
_BCAST = np.zeros((_E, _E * _D), np.float32)
_SUM = np.zeros((_E * _D, _D), np.float32)
for e in range(_E):
    for o in range(_D):
        _BCAST[e, e * _D + o] = 1.0
        _SUM[e * _D + o, o] = 1.0


def _moe_block(x_ref, gw_ref, wcat_ref, dif_ref, m_ref, bcast_ref, sum_ref,
               out_ref):
    xb = x_ref[...]
    logits = jax.lax.dot_general(xb, gw_ref[...], _CONTRACT_1_1,
                                 preferred_element_type=jnp.float32)
    y = jax.lax.dot_general(xb, wcat_ref[...], _CONTRACT_1_1,
                            preferred_element_type=jnp.float32)
    d = jax.lax.dot_general(logits, dif_ref[...], _CONTRACT_1_0,
                            preferred_element_type=jnp.float32)
    c = (d > 0).astype(jnp.float32)
    bm = jax.lax.dot_general(c, m_ref[...], _CONTRACT_1_0,
                             preferred_element_type=jnp.float32)
    lane = jax.lax.broadcasted_iota(jnp.int32, bm.shape, 1).astype(jnp.float32)
    mask = (bm + lane < 1.5).astype(jnp.float32)
    maskf = jax.lax.dot_general(mask, bcast_ref[...], _CONTRACT_1_0,
                                preferred_element_type=jnp.float32)
    out_ref[...] = jax.lax.dot_general(maskf * y, sum_ref[...], _CONTRACT_1_0,
                                       preferred_element_type=jnp.float32)


@jax.jit
def kernel(x, gate_w, expert_w):
    wcat = expert_w.reshape(_E * _D, _D)
    grid = (_TOKENS // _BLK,)
    full = lambda a: pl.BlockSpec(a.shape, lambda i: (0,) * a.ndim)
    consts = (jnp.asarray(_DIF), jnp.asarray(_M), jnp.asarray(_BCAST),
              jnp.asarray(_SUM))
    return pl.pallas_call(
        _moe_block,
        grid=grid,
        in_specs=[
            pl.BlockSpec((_BLK, _D), lambda i: (i, 0)),
            full(gate_w), full(wcat),
            *[full(c) for c in consts],
        ],
        out_specs=pl.BlockSpec((_BLK, _D), lambda i: (i, 0)),
        out_shape=jax.ShapeDtypeStruct((_TOKENS, _D), jnp.float32),
    )(x, gate_w, wcat, *consts)
